# baseline (speedup 1.0000x reference)
import functools

import jax
import jax.numpy as jnp
from jax import lax
from jax.experimental import pallas as pl
from jax.experimental.pallas import tpu as pltpu
from jax.experimental.pallas import tpu_sc as plsc

B = 1_000_000
D = 128
V = 83
NC, NS = 2, 16
NW = NC * NS
W = 31_248
SUB = 56
N_SUB = W // SUB
NBUF = 9
GROUPS = N_SUB // NBUF
GA = 5
SL = 4
TAIL_BASE = NW * W
TAIL = B - TAIL_BASE

_mesh = plsc.VectorSubcoreMesh(core_axis_name="c", subcore_axis_name="s")


@functools.partial(
    pl.kernel,
    out_type=jax.ShapeDtypeStruct((B, D), jnp.float32),
    mesh=_mesh,
    scratch_types=[
        pltpu.VMEM((W,), jnp.int32),
        [pltpu.VMEM((SUB, D), jnp.float32) for _ in range(NBUF)],
        [pltpu.SemaphoreType.DMA for _ in range(NBUF)],
        [pltpu.SemaphoreType.DMA for _ in range(NBUF)],
        pltpu.VMEM((TAIL,), jnp.int32),
        pltpu.VMEM((TAIL, D), jnp.float32),
        pltpu.SemaphoreType.DMA,
        pltpu.VMEM_SHARED((NS * V, D), jnp.float32),
    ],
)
def _gather_kernel(idx_hbm, table_hbm, out_hbm, idx_v, bufs, sg, ss,
                   tidx_v, trows_v, tsem, table_sh):
    wid = lax.axis_index("s") * NC + lax.axis_index("c")
    base = wid * W

    sid = lax.axis_index("s")
    pltpu.sync_copy(table_hbm, table_sh.at[pl.ds(sid * V, V)])
    plsc.subcore_barrier()

    pltpu.sync_copy(idx_hbm.at[pl.ds(base, W)], idx_v)

    off = jnp.full((16,), sid * V, jnp.int32)

    def adj(k, carry):
        idx_v[pl.ds(k * 16, 16)] = idx_v[pl.ds(k * 16, 16)] + off
        return carry

    lax.fori_loop(0, W // 16, adj, 0)

    def g_copy(j, b):
        return pltpu.make_async_copy(
            table_sh.at[idx_v.at[pl.ds(j * SUB, SUB)]], bufs[b], sg[b])

    def s_copy(j, b):
        return pltpu.make_async_copy(
            bufs[b], out_hbm.at[pl.ds(base + j * SUB, SUB)], ss[b])

    for j in range(GA):
        g_copy(j, j % NBUF).start()

    def step(b, jj):
        @pl.when(jj >= SL)
        def _():
            s_copy(jj - SL, (b - SL) % NBUF).wait()

        @pl.when(jj + GA < N_SUB)
        def _():
            g_copy(jj + GA, (b + GA) % NBUF).start()

        g_copy(jj, b).wait()
        s_copy(jj, b).start()

    def group(jo, carry):
        for b in range(NBUF):
            step(b, jo * NBUF + b)
        return carry

    lax.fori_loop(0, GROUPS, group, 0)

    for j in range(N_SUB - SL, N_SUB):
        s_copy(j, j % NBUF).wait()

    @pl.when(wid == 0)
    def _():
        pltpu.sync_copy(idx_hbm.at[pl.ds(TAIL_BASE, TAIL)], tidx_v)
        pltpu.async_copy(table_sh.at[tidx_v], trows_v, tsem).wait()
        pltpu.sync_copy(trows_v, out_hbm.at[pl.ds(TAIL_BASE, TAIL)])


def kernel(atom_number, embedding_list):
    return _gather_kernel(atom_number, embedding_list)


# speedup vs baseline: 1.0529x; 1.0529x over previous
import functools

import jax
import jax.numpy as jnp
from jax import lax
from jax.experimental import pallas as pl
from jax.experimental.pallas import tpu as pltpu
from jax.experimental.pallas import tpu_sc as plsc

B = 1_000_000
D = 128
V = 83
NC, NS = 2, 16
NW = NC * NS
W = 31_248
SUB = 56
N_SUB = W // SUB
NBUF = 6
GROUPS = N_SUB // NBUF
GA = 4
SL = 2
TAIL_BASE = NW * W
TAIL = B - TAIL_BASE

_mesh = plsc.VectorSubcoreMesh(core_axis_name="c", subcore_axis_name="s")


@functools.partial(
    pl.kernel,
    out_type=jax.ShapeDtypeStruct((B, D), jnp.float32),
    mesh=_mesh,
    scratch_types=[
        pltpu.VMEM((W,), jnp.int32),
        [pltpu.VMEM((SUB, D), jnp.float32) for _ in range(NBUF)],
        [pltpu.SemaphoreType.DMA for _ in range(NBUF)],
        [pltpu.SemaphoreType.DMA for _ in range(NBUF)],
        pltpu.VMEM((TAIL,), jnp.int32),
        pltpu.VMEM((TAIL, D), jnp.float32),
        pltpu.SemaphoreType.DMA,
        pltpu.VMEM_SHARED((V, D), jnp.float32),
    ],
)
def _gather_kernel(idx_hbm, table_hbm, out_hbm, idx_v, bufs, sg, ss,
                   tidx_v, trows_v, tsem, table_sh):
    wid = lax.axis_index("s") * NC + lax.axis_index("c")
    base = wid * W

    @pl.when(lax.axis_index("s") == 0)
    def _():
        pltpu.sync_copy(table_hbm, table_sh)

    plsc.subcore_barrier()

    pltpu.sync_copy(idx_hbm.at[pl.ds(base, W)], idx_v)

    def g_copy(j, b):
        return pltpu.make_async_copy(
            table_sh.at[idx_v.at[pl.ds(j * SUB, SUB)]], bufs[b], sg[b])

    def s_copy(j, b):
        return pltpu.make_async_copy(
            bufs[b], out_hbm.at[pl.ds(base + j * SUB, SUB)], ss[b])

    for j in range(GA):
        g_copy(j, j % NBUF).start()

    def step(b, jj):
        @pl.when(jj >= SL)
        def _():
            s_copy(jj - SL, (b - SL) % NBUF).wait()

        @pl.when(jj + GA < N_SUB)
        def _():
            g_copy(jj + GA, (b + GA) % NBUF).start()

        g_copy(jj, b).wait()
        s_copy(jj, b).start()

    def group(jo, carry):
        for b in range(NBUF):
            step(b, jo * NBUF + b)
        return carry

    lax.fori_loop(0, GROUPS, group, 0)

    for j in range(N_SUB - SL, N_SUB):
        s_copy(j, j % NBUF).wait()

    @pl.when(wid == 0)
    def _():
        pltpu.sync_copy(idx_hbm.at[pl.ds(TAIL_BASE, TAIL)], tidx_v)
        pltpu.async_copy(table_sh.at[tidx_v], trows_v, tsem).wait()
        pltpu.sync_copy(trows_v, out_hbm.at[pl.ds(TAIL_BASE, TAIL)])


def kernel(atom_number, embedding_list):
    return _gather_kernel(atom_number, embedding_list)
